# baseline (speedup 1.0000x reference)
import functools

import jax
import jax.numpy as jnp
from jax import lax
from jax.experimental import pallas as pl
from jax.experimental.pallas import tpu as pltpu
from jax.experimental.pallas import tpu_sc as plsc

BN_EPS = 1e-5

NC = 2
NS = 16
NW = NC * NS


def _mm_body(x_ref, w_ref, y_ref):
    y_ref[...] = jnp.dot(x_ref[...], w_ref[0], preferred_element_type=jnp.float32)


def _transform_features(x, W):
    n, c_in = x.shape
    k, _, c_out = W.shape
    return pl.pallas_call(
        _mm_body,
        grid=(k,),
        in_specs=[
            pl.BlockSpec((n, c_in), lambda i: (0, 0)),
            pl.BlockSpec((1, c_in, c_out), lambda i: (i, 0, 0)),
        ],
        out_specs=pl.BlockSpec((n, c_out), lambda i: (i, 0)),
        out_shape=jax.ShapeDtypeStruct((k * n, c_out), jnp.float32),
    )(x, W)


def _edge_prep_body(n, ei_ref, ko_ref, g_ref, d_ref):
    g_ref[...] = ko_ref[...] * n + ei_ref[0, :]
    d_ref[...] = ei_ref[1, :]


def _edge_prep(edge_index, kernel_offsets, n):
    e = edge_index.shape[1]
    return pl.pallas_call(
        functools.partial(_edge_prep_body, n),
        in_specs=[
            pl.BlockSpec((2, e), lambda: (0, 0)),
            pl.BlockSpec((e,), lambda: (0,)),
        ],
        out_specs=[
            pl.BlockSpec((e,), lambda: (0,)),
            pl.BlockSpec((e,), lambda: (0,)),
        ],
        out_shape=[
            jax.ShapeDtypeStruct((e,), jnp.int32),
            jax.ShapeDtypeStruct((e,), jnp.int32),
        ],
    )(edge_index, kernel_offsets)


def _make_sc_edge_kernel(n, c, e, ch, b):
    mesh = plsc.VectorSubcoreMesh(
        core_axis_name="c", subcore_axis_name="s", num_cores=NC, num_subcores=NS
    )
    r8 = (n // NS) // 8 * 8
    r_last = n - (NS - 1) * r8
    e_per_w = e // NW

    @functools.partial(
        pl.kernel,
        mesh=mesh,
        out_type=jax.ShapeDtypeStruct((NC, n, c), jnp.float32),
        scratch_types=[
            pltpu.VMEM_SHARED((n, c), jnp.float32),
            pltpu.VMEM((e_per_w,), jnp.int32),
            pltpu.VMEM((b,), jnp.int32),
            pltpu.VMEM((b,), jnp.int32),
            pltpu.VMEM((b,), jnp.int32),
            pltpu.VMEM((b, c), jnp.float32),
            pltpu.VMEM((b, c), jnp.float32),
            pltpu.VMEM((b, c), jnp.float32),
            pltpu.SemaphoreType.DMA,
            pltpu.SemaphoreType.DMA,
            pltpu.SemaphoreType.DMA,
            pltpu.SemaphoreType.DMA,
            pltpu.SemaphoreType.DMA,
            pltpu.SemaphoreType.DMA,
            pltpu.SemaphoreType.DMA,
            pltpu.SemaphoreType.DMA,
            pltpu.SemaphoreType.DMA,
        ],
    )
    def sc_kernel(y_hbm, gidx_hbm, dst_hbm, zeros_hbm, part_hbm,
                  acc, gidx_v, dst_a, dst_b, dst_c, rows_a, rows_b, rows_c,
                  sem_ra, sem_rb, sem_rc, sem_da, sem_db, sem_dc,
                  sem_sa, sem_sb, sem_sc):
        cid = lax.axis_index("c")
        sid = lax.axis_index("s")
        wid = cid * NS + sid
        base = wid * e_per_w

        pltpu.sync_copy(gidx_hbm.at[pl.ds(base, e_per_w)], gidx_v)

        def chunk(g, dst_buf, rows_buf, sem_r, sem_d):
            d_d = pltpu.async_copy(dst_hbm.at[pl.ds(base + g * b, b)], dst_buf, sem_d)
            d_r = pltpu.async_copy(y_hbm.at[gidx_v.at[pl.ds(g * b, b)]], rows_buf, sem_r)
            return d_d, d_r

        def drain(pair):
            d_d, d_r = pair
            d_r.wait()
            d_d.wait()

        in_a0 = chunk(0, dst_a, rows_a, sem_ra, sem_da)
        in_b0 = chunk(1, dst_b, rows_b, sem_rb, sem_db)
        in_c0 = chunk(2, dst_c, rows_c, sem_rc, sem_dc)

        @pl.when(sid < NS - 1)
        def _():
            pltpu.sync_copy(zeros_hbm.at[pl.ds(0, r8)], acc.at[pl.ds(sid * r8, r8)])

        @pl.when(sid == NS - 1)
        def _():
            pltpu.sync_copy(zeros_hbm, acc.at[pl.ds((NS - 1) * r8, r_last)])

        plsc.subcore_barrier()

        def scatter_triple(in_a, in_b, in_c):
            drain(in_a)
            s_a = pltpu.async_copy(rows_a, acc.at[dst_a], sem_sa, add=True)
            drain(in_b)
            s_b = pltpu.async_copy(rows_b, acc.at[dst_b], sem_sb, add=True)
            drain(in_c)
            s_c = pltpu.async_copy(rows_c, acc.at[dst_c], sem_sc, add=True)
            s_a.wait()
            s_b.wait()
            s_c.wait()

        scatter_triple(in_a0, in_b0, in_c0)

        def triple_body(p, carry):
            g = p * 3
            in_a = chunk(g, dst_a, rows_a, sem_ra, sem_da)
            in_b = chunk(g + 1, dst_b, rows_b, sem_rb, sem_db)
            in_c = chunk(g + 2, dst_c, rows_c, sem_rc, sem_dc)
            scatter_triple(in_a, in_b, in_c)
            return carry

        lax.fori_loop(1, ch // 3, triple_body, 0)
        for g in range((ch // 3) * 3, ch):
            in_t = chunk(g, dst_a, rows_a, sem_ra, sem_da)
            drain(in_t)
            pltpu.sync_copy(rows_a, acc.at[dst_a], add=True)
        plsc.subcore_barrier()

        @pl.when(sid < NS - 1)
        def _():
            pltpu.sync_copy(
                acc.at[pl.ds(sid * r8, r8)],
                part_hbm.at[cid].at[pl.ds(sid * r8, r8)],
            )

        @pl.when(sid == NS - 1)
        def _():
            pltpu.sync_copy(
                acc.at[pl.ds((NS - 1) * r8, r_last)],
                part_hbm.at[cid].at[pl.ds((NS - 1) * r8, r_last)],
            )

    return sc_kernel


def _combine_bn_body(n, p_ref, g_ref, b_ref, o_ref):
    s = p_ref[0] + p_ref[1]
    mean = jnp.sum(s, axis=0, keepdims=True) / n
    var = jnp.sum(s * s, axis=0, keepdims=True) / n - mean * mean
    scale = lax.rsqrt(var + BN_EPS) * g_ref[...]
    o_ref[...] = (s - mean) * scale + b_ref[...]


def _combine_bn(part, gamma, beta):
    _, n, c = part.shape
    return pl.pallas_call(
        functools.partial(_combine_bn_body, n),
        in_specs=[
            pl.BlockSpec((2, n, c), lambda: (0, 0, 0)),
            pl.BlockSpec((1, c), lambda: (0, 0)),
            pl.BlockSpec((1, c), lambda: (0, 0)),
        ],
        out_specs=pl.BlockSpec((n, c), lambda: (0, 0)),
        out_shape=jax.ShapeDtypeStruct((n, c), jnp.float32),
    )(part, gamma, beta)


@jax.jit
def kernel(x, edge_index, kernel_offsets, W, bn_gamma, bn_beta):
    n, c_in = x.shape
    k, _, c_out = W.shape
    e = edge_index.shape[1]

    y = _transform_features(x, W)

    gidx, dst = _edge_prep(edge_index, kernel_offsets, n)

    b = 40
    ch = e // (NW * b)
    r8 = (n // NS) // 8 * 8
    zeros = jnp.zeros((n - (NS - 1) * r8, c_out), jnp.float32)

    part = _make_sc_edge_kernel(n, c_out, e, ch, b)(y, gidx, dst, zeros)

    return _combine_bn(part, bn_gamma.reshape(1, c_out),
                       bn_beta.reshape(1, c_out))


# speedup vs baseline: 1.0983x; 1.0983x over previous
import functools

import jax
import jax.numpy as jnp
from jax import lax
from jax.experimental import pallas as pl
from jax.experimental.pallas import tpu as pltpu
from jax.experimental.pallas import tpu_sc as plsc

BN_EPS = 1e-5

NC = 2
NS = 16
NW = NC * NS


def _mm_prep_body(n, x_ref, w_ref, ei_ref, ko_ref, y_ref, g_ref, d_ref, z_ref):
    y_ref[...] = jnp.dot(x_ref[...], w_ref[0], preferred_element_type=jnp.float32)

    @pl.when(pl.program_id(0) == 0)
    def _():
        g_ref[...] = ko_ref[...] * n + ei_ref[0, :]
        d_ref[...] = ei_ref[1, :]
        z_ref[...] = jnp.zeros_like(z_ref)


def _transform_features(x, W, edge_index, kernel_offsets, zrows):
    n, c_in = x.shape
    k, _, c_out = W.shape
    e = edge_index.shape[1]
    const1 = lambda i: (0,)
    const2 = lambda i: (0, 0)
    return pl.pallas_call(
        functools.partial(_mm_prep_body, n),
        grid=(k,),
        in_specs=[
            pl.BlockSpec((n, c_in), const2),
            pl.BlockSpec((1, c_in, c_out), lambda i: (i, 0, 0)),
            pl.BlockSpec((2, e), const2),
            pl.BlockSpec((e,), const1),
        ],
        out_specs=[
            pl.BlockSpec((n, c_out), lambda i: (i, 0)),
            pl.BlockSpec((e,), const1),
            pl.BlockSpec((e,), const1),
            pl.BlockSpec((zrows, c_out), const2),
        ],
        out_shape=[
            jax.ShapeDtypeStruct((k * n, c_out), jnp.float32),
            jax.ShapeDtypeStruct((e,), jnp.int32),
            jax.ShapeDtypeStruct((e,), jnp.int32),
            jax.ShapeDtypeStruct((zrows, c_out), jnp.float32),
        ],
    )(x, W, edge_index, kernel_offsets)


def _make_sc_edge_kernel(n, c, e, ch, b):
    mesh = plsc.VectorSubcoreMesh(
        core_axis_name="c", subcore_axis_name="s", num_cores=NC, num_subcores=NS
    )
    r8 = (n // NS) // 8 * 8
    r_last = n - (NS - 1) * r8
    e_per_w = e // NW

    @functools.partial(
        pl.kernel,
        mesh=mesh,
        out_type=jax.ShapeDtypeStruct((NC, n, c), jnp.float32),
        scratch_types=[
            pltpu.VMEM_SHARED((n, c), jnp.float32),
            pltpu.VMEM((e_per_w,), jnp.int32),
            pltpu.VMEM((b,), jnp.int32),
            pltpu.VMEM((b,), jnp.int32),
            pltpu.VMEM((b,), jnp.int32),
            pltpu.VMEM((b, c), jnp.float32),
            pltpu.VMEM((b, c), jnp.float32),
            pltpu.VMEM((b, c), jnp.float32),
            pltpu.SemaphoreType.DMA,
            pltpu.SemaphoreType.DMA,
            pltpu.SemaphoreType.DMA,
            pltpu.SemaphoreType.DMA,
            pltpu.SemaphoreType.DMA,
            pltpu.SemaphoreType.DMA,
            pltpu.SemaphoreType.DMA,
            pltpu.SemaphoreType.DMA,
            pltpu.SemaphoreType.DMA,
        ],
    )
    def sc_kernel(y_hbm, gidx_hbm, dst_hbm, zeros_hbm, part_hbm,
                  acc, gidx_v, dst_a, dst_b, dst_c, rows_a, rows_b, rows_c,
                  sem_ra, sem_rb, sem_rc, sem_da, sem_db, sem_dc,
                  sem_sa, sem_sb, sem_sc):
        cid = lax.axis_index("c")
        sid = lax.axis_index("s")
        wid = cid * NS + sid
        base = wid * e_per_w

        pltpu.sync_copy(gidx_hbm.at[pl.ds(base, e_per_w)], gidx_v)

        def chunk(g, dst_buf, rows_buf, sem_r, sem_d):
            d_d = pltpu.async_copy(dst_hbm.at[pl.ds(base + g * b, b)], dst_buf, sem_d)
            d_r = pltpu.async_copy(y_hbm.at[gidx_v.at[pl.ds(g * b, b)]], rows_buf, sem_r)
            return d_d, d_r

        def drain(pair):
            d_d, d_r = pair
            d_r.wait()
            d_d.wait()

        in_a0 = chunk(0, dst_a, rows_a, sem_ra, sem_da)
        in_b0 = chunk(1, dst_b, rows_b, sem_rb, sem_db)
        in_c0 = chunk(2, dst_c, rows_c, sem_rc, sem_dc)

        @pl.when(sid < NS - 1)
        def _():
            pltpu.sync_copy(zeros_hbm.at[pl.ds(0, r8)], acc.at[pl.ds(sid * r8, r8)])

        @pl.when(sid == NS - 1)
        def _():
            pltpu.sync_copy(zeros_hbm, acc.at[pl.ds((NS - 1) * r8, r_last)])

        plsc.subcore_barrier()

        def scatter_triple(in_a, in_b, in_c):
            drain(in_a)
            s_a = pltpu.async_copy(rows_a, acc.at[dst_a], sem_sa, add=True)
            drain(in_b)
            s_b = pltpu.async_copy(rows_b, acc.at[dst_b], sem_sb, add=True)
            drain(in_c)
            s_c = pltpu.async_copy(rows_c, acc.at[dst_c], sem_sc, add=True)
            s_a.wait()
            s_b.wait()
            s_c.wait()

        scatter_triple(in_a0, in_b0, in_c0)

        def triple_body(p, carry):
            g = p * 3
            in_a = chunk(g, dst_a, rows_a, sem_ra, sem_da)
            in_b = chunk(g + 1, dst_b, rows_b, sem_rb, sem_db)
            in_c = chunk(g + 2, dst_c, rows_c, sem_rc, sem_dc)
            scatter_triple(in_a, in_b, in_c)
            return carry

        lax.fori_loop(1, ch // 3, triple_body, 0)
        for g in range((ch // 3) * 3, ch):
            in_t = chunk(g, dst_a, rows_a, sem_ra, sem_da)
            drain(in_t)
            pltpu.sync_copy(rows_a, acc.at[dst_a], add=True)
        plsc.subcore_barrier()

        @pl.when(sid < NS - 1)
        def _():
            pltpu.sync_copy(
                acc.at[pl.ds(sid * r8, r8)],
                part_hbm.at[cid].at[pl.ds(sid * r8, r8)],
            )

        @pl.when(sid == NS - 1)
        def _():
            pltpu.sync_copy(
                acc.at[pl.ds((NS - 1) * r8, r_last)],
                part_hbm.at[cid].at[pl.ds((NS - 1) * r8, r_last)],
            )

    return sc_kernel


def _combine_bn_body(n, p_ref, g_ref, b_ref, o_ref):
    s = p_ref[0] + p_ref[1]
    mean = jnp.sum(s, axis=0, keepdims=True) / n
    var = jnp.sum(s * s, axis=0, keepdims=True) / n - mean * mean
    scale = lax.rsqrt(var + BN_EPS) * g_ref[...]
    o_ref[...] = (s - mean) * scale + b_ref[...]


def _combine_bn(part, gamma, beta):
    _, n, c = part.shape
    return pl.pallas_call(
        functools.partial(_combine_bn_body, n),
        in_specs=[
            pl.BlockSpec((2, n, c), lambda: (0, 0, 0)),
            pl.BlockSpec((1, c), lambda: (0, 0)),
            pl.BlockSpec((1, c), lambda: (0, 0)),
        ],
        out_specs=pl.BlockSpec((n, c), lambda: (0, 0)),
        out_shape=jax.ShapeDtypeStruct((n, c), jnp.float32),
    )(part, gamma, beta)


@jax.jit
def kernel(x, edge_index, kernel_offsets, W, bn_gamma, bn_beta):
    n, c_in = x.shape
    k, _, c_out = W.shape
    e = edge_index.shape[1]

    b = 80
    ch = e // (NW * b)
    r8 = (n // NS) // 8 * 8

    y, gidx, dst, zeros = _transform_features(
        x, W, edge_index, kernel_offsets, n - (NS - 1) * r8)

    part = _make_sc_edge_kernel(n, c_out, e, ch, b)(y, gidx, dst, zeros)

    return _combine_bn(part, bn_gamma.reshape(1, c_out),
                       bn_beta.reshape(1, c_out))


# speedup vs baseline: 1.2105x; 1.1021x over previous
import functools

import jax
import jax.numpy as jnp
from jax import lax
from jax.experimental import pallas as pl
from jax.experimental.pallas import tpu as pltpu
from jax.experimental.pallas import tpu_sc as plsc

BN_EPS = 1e-5

NC = 2
NS = 16
NW = NC * NS


def _mm_prep_body(n, x_ref, w_ref, ei_ref, ko_ref, y_ref, g_ref, d_ref, z_ref):
    y_ref[...] = jnp.dot(x_ref[...], w_ref[0], preferred_element_type=jnp.float32)

    @pl.when(pl.program_id(0) == 0)
    def _():
        g_ref[...] = ko_ref[...] * n + ei_ref[0, :]
        d_ref[...] = ei_ref[1, :]
        z_ref[...] = jnp.zeros_like(z_ref)


def _transform_features(x, W, edge_index, kernel_offsets, zrows):
    n, c_in = x.shape
    k, _, c_out = W.shape
    e = edge_index.shape[1]
    const1 = lambda i: (0,)
    const2 = lambda i: (0, 0)
    return pl.pallas_call(
        functools.partial(_mm_prep_body, n),
        grid=(k,),
        in_specs=[
            pl.BlockSpec((n, c_in), const2),
            pl.BlockSpec((1, c_in, c_out), lambda i: (i, 0, 0)),
            pl.BlockSpec((2, e), const2),
            pl.BlockSpec((e,), const1),
        ],
        out_specs=[
            pl.BlockSpec((n, c_out), lambda i: (i, 0)),
            pl.BlockSpec((e,), const1),
            pl.BlockSpec((e,), const1),
            pl.BlockSpec((zrows, c_out), const2),
        ],
        out_shape=[
            jax.ShapeDtypeStruct((k * n, c_out), jnp.float32),
            jax.ShapeDtypeStruct((e,), jnp.int32),
            jax.ShapeDtypeStruct((e,), jnp.int32),
            jax.ShapeDtypeStruct((zrows, c_out), jnp.float32),
        ],
    )(x, W, edge_index, kernel_offsets)


def _make_sc_edge_kernel(n, c, e, ch, b):
    mesh = plsc.VectorSubcoreMesh(
        core_axis_name="c", subcore_axis_name="s", num_cores=NC, num_subcores=NS
    )
    r8 = (n // NS) // 8 * 8
    r_last = n - (NS - 1) * r8
    e_per_w = e // NW

    @functools.partial(
        pl.kernel,
        mesh=mesh,
        out_type=jax.ShapeDtypeStruct((NC, n, c), jnp.float32),
        scratch_types=[
            pltpu.VMEM_SHARED((n, c), jnp.float32),
            pltpu.VMEM((e_per_w,), jnp.int32),
            pltpu.VMEM((b,), jnp.int32),
            pltpu.VMEM((b,), jnp.int32),
            pltpu.VMEM((b,), jnp.int32),
            pltpu.VMEM((b, c), jnp.float32),
            pltpu.VMEM((b, c), jnp.float32),
            pltpu.VMEM((b, c), jnp.float32),
            pltpu.SemaphoreType.DMA,
            pltpu.SemaphoreType.DMA,
            pltpu.SemaphoreType.DMA,
            pltpu.SemaphoreType.DMA,
            pltpu.SemaphoreType.DMA,
            pltpu.SemaphoreType.DMA,
            pltpu.SemaphoreType.DMA,
            pltpu.SemaphoreType.DMA,
            pltpu.SemaphoreType.DMA,
        ],
    )
    def sc_kernel(y_hbm, gidx_hbm, dst_hbm, zeros_hbm, part_hbm,
                  acc, gidx_v, dst_a, dst_b, dst_c, rows_a, rows_b, rows_c,
                  sem_ra, sem_rb, sem_rc, sem_da, sem_db, sem_dc,
                  sem_sa, sem_sb, sem_sc):
        cid = lax.axis_index("c")
        sid = lax.axis_index("s")
        wid = cid * NS + sid
        base = wid * e_per_w

        pltpu.sync_copy(gidx_hbm.at[pl.ds(base, e_per_w)], gidx_v)

        def chunk(g, dst_buf, rows_buf, sem_r, sem_d):
            d_d = pltpu.async_copy(dst_hbm.at[pl.ds(base + g * b, b)], dst_buf, sem_d)
            d_r = pltpu.async_copy(y_hbm.at[gidx_v.at[pl.ds(g * b, b)]], rows_buf, sem_r)
            return d_d, d_r

        def drain(pair):
            d_d, d_r = pair
            d_r.wait()
            d_d.wait()

        in_a0 = chunk(0, dst_a, rows_a, sem_ra, sem_da)
        in_b0 = chunk(1, dst_b, rows_b, sem_rb, sem_db)
        in_c0 = chunk(2, dst_c, rows_c, sem_rc, sem_dc)

        @pl.when(sid < NS - 1)
        def _():
            pltpu.sync_copy(zeros_hbm.at[pl.ds(0, r8)], acc.at[pl.ds(sid * r8, r8)])

        @pl.when(sid == NS - 1)
        def _():
            pltpu.sync_copy(zeros_hbm, acc.at[pl.ds((NS - 1) * r8, r_last)])

        plsc.subcore_barrier()

        del in_a0, in_b0, in_c0

        def wait_gather(g, dst_buf, rows_buf, sem_r, sem_d):
            pltpu.make_async_copy(
                y_hbm.at[gidx_v.at[pl.ds(g * b, b)]], rows_buf, sem_r).wait()
            pltpu.make_async_copy(
                dst_hbm.at[pl.ds(base + g * b, b)], dst_buf, sem_d).wait()

        nt = ch // 3

        def triple_body(p, carry):
            g = p * 3
            wait_gather(g, dst_a, rows_a, sem_ra, sem_da)
            s_a = pltpu.async_copy(rows_a, acc.at[dst_a], sem_sa, add=True)
            wait_gather(g + 1, dst_b, rows_b, sem_rb, sem_db)
            s_b = pltpu.async_copy(rows_b, acc.at[dst_b], sem_sb, add=True)
            wait_gather(g + 2, dst_c, rows_c, sem_rc, sem_dc)
            s_c = pltpu.async_copy(rows_c, acc.at[dst_c], sem_sc, add=True)
            s_a.wait()
            chunk(g + 3, dst_a, rows_a, sem_ra, sem_da)
            s_b.wait()
            chunk(g + 4, dst_b, rows_b, sem_rb, sem_db)
            s_c.wait()
            chunk(g + 5, dst_c, rows_c, sem_rc, sem_dc)
            return carry

        lax.fori_loop(0, nt - 1, triple_body, 0)

        gl = (nt - 1) * 3
        wait_gather(gl, dst_a, rows_a, sem_ra, sem_da)
        s_a = pltpu.async_copy(rows_a, acc.at[dst_a], sem_sa, add=True)
        wait_gather(gl + 1, dst_b, rows_b, sem_rb, sem_db)
        s_b = pltpu.async_copy(rows_b, acc.at[dst_b], sem_sb, add=True)
        wait_gather(gl + 2, dst_c, rows_c, sem_rc, sem_dc)
        s_c = pltpu.async_copy(rows_c, acc.at[dst_c], sem_sc, add=True)
        s_a.wait()
        s_b.wait()
        s_c.wait()
        for g in range((ch // 3) * 3, ch):
            in_t = chunk(g, dst_a, rows_a, sem_ra, sem_da)
            drain(in_t)
            pltpu.sync_copy(rows_a, acc.at[dst_a], add=True)
        plsc.subcore_barrier()

        @pl.when(sid < NS - 1)
        def _():
            pltpu.sync_copy(
                acc.at[pl.ds(sid * r8, r8)],
                part_hbm.at[cid].at[pl.ds(sid * r8, r8)],
            )

        @pl.when(sid == NS - 1)
        def _():
            pltpu.sync_copy(
                acc.at[pl.ds((NS - 1) * r8, r_last)],
                part_hbm.at[cid].at[pl.ds((NS - 1) * r8, r_last)],
            )

    return sc_kernel


def _combine_bn_body(n, p_ref, g_ref, b_ref, o_ref):
    s = p_ref[0] + p_ref[1]
    mean = jnp.sum(s, axis=0, keepdims=True) / n
    var = jnp.sum(s * s, axis=0, keepdims=True) / n - mean * mean
    scale = lax.rsqrt(var + BN_EPS) * g_ref[...]
    o_ref[...] = (s - mean) * scale + b_ref[...]


def _combine_bn(part, gamma, beta):
    _, n, c = part.shape
    return pl.pallas_call(
        functools.partial(_combine_bn_body, n),
        in_specs=[
            pl.BlockSpec((2, n, c), lambda: (0, 0, 0)),
            pl.BlockSpec((1, c), lambda: (0, 0)),
            pl.BlockSpec((1, c), lambda: (0, 0)),
        ],
        out_specs=pl.BlockSpec((n, c), lambda: (0, 0)),
        out_shape=jax.ShapeDtypeStruct((n, c), jnp.float32),
    )(part, gamma, beta)


@jax.jit
def kernel(x, edge_index, kernel_offsets, W, bn_gamma, bn_beta):
    n, c_in = x.shape
    k, _, c_out = W.shape
    e = edge_index.shape[1]

    b = 80
    ch = e // (NW * b)
    r8 = (n // NS) // 8 * 8

    y, gidx, dst, zeros = _transform_features(
        x, W, edge_index, kernel_offsets, n - (NS - 1) * r8)

    part = _make_sc_edge_kernel(n, c_out, e, ch, b)(y, gidx, dst, zeros)

    return _combine_bn(part, bn_gamma.reshape(1, c_out),
                       bn_beta.reshape(1, c_out))


# speedup vs baseline: 1.2251x; 1.0121x over previous
import functools

import jax
import jax.numpy as jnp
from jax import lax
from jax.experimental import pallas as pl
from jax.experimental.pallas import tpu as pltpu
from jax.experimental.pallas import tpu_sc as plsc

BN_EPS = 1e-5

NC = 2
NS = 16
NW = NC * NS


def _mm_prep_body(n, x_ref, w_ref, ei_ref, ko_ref, y_ref, g_ref, d_ref, z_ref):
    y_ref[...] = jnp.dot(x_ref[...], w_ref[0], preferred_element_type=jnp.float32)

    @pl.when(pl.program_id(0) == 0)
    def _():
        g_ref[...] = ko_ref[...] * n + ei_ref[0, :]
        d_ref[...] = ei_ref[1, :]
        z_ref[...] = jnp.zeros_like(z_ref)


def _transform_features(x, W, edge_index, kernel_offsets, zrows):
    n, c_in = x.shape
    k, _, c_out = W.shape
    e = edge_index.shape[1]
    const1 = lambda i: (0,)
    const2 = lambda i: (0, 0)
    return pl.pallas_call(
        functools.partial(_mm_prep_body, n),
        grid=(k,),
        in_specs=[
            pl.BlockSpec((n, c_in), const2),
            pl.BlockSpec((1, c_in, c_out), lambda i: (i, 0, 0)),
            pl.BlockSpec((2, e), const2),
            pl.BlockSpec((e,), const1),
        ],
        out_specs=[
            pl.BlockSpec((n, c_out), lambda i: (i, 0)),
            pl.BlockSpec((e,), const1),
            pl.BlockSpec((e,), const1),
            pl.BlockSpec((zrows, c_out), const2),
        ],
        out_shape=[
            jax.ShapeDtypeStruct((k * n, c_out), jnp.float32),
            jax.ShapeDtypeStruct((e,), jnp.int32),
            jax.ShapeDtypeStruct((e,), jnp.int32),
            jax.ShapeDtypeStruct((zrows, c_out), jnp.float32),
        ],
    )(x, W, edge_index, kernel_offsets)


def _make_sc_edge_kernel(n, c, e, ch, b, bt):
    mesh = plsc.VectorSubcoreMesh(
        core_axis_name="c", subcore_axis_name="s", num_cores=NC, num_subcores=NS
    )
    r8 = (n // NS) // 8 * 8
    r_last = n - (NS - 1) * r8
    e_per_w = e // NW

    @functools.partial(
        pl.kernel,
        mesh=mesh,
        out_type=jax.ShapeDtypeStruct((NC, n, c), jnp.float32),
        scratch_types=[
            pltpu.VMEM_SHARED((n, c), jnp.float32),
            pltpu.VMEM((e_per_w,), jnp.int32),
            pltpu.VMEM((b,), jnp.int32),
            pltpu.VMEM((b,), jnp.int32),
            pltpu.VMEM((b,), jnp.int32),
            pltpu.VMEM((b, c), jnp.float32),
            pltpu.VMEM((b, c), jnp.float32),
            pltpu.VMEM((b, c), jnp.float32),
            pltpu.VMEM((bt,), jnp.int32),
            pltpu.VMEM((bt, c), jnp.float32),
            pltpu.SemaphoreType.DMA,
            pltpu.SemaphoreType.DMA,
            pltpu.SemaphoreType.DMA,
            pltpu.SemaphoreType.DMA,
            pltpu.SemaphoreType.DMA,
            pltpu.SemaphoreType.DMA,
            pltpu.SemaphoreType.DMA,
            pltpu.SemaphoreType.DMA,
            pltpu.SemaphoreType.DMA,
            pltpu.SemaphoreType.DMA,
            pltpu.SemaphoreType.DMA,
        ],
    )
    def sc_kernel(y_hbm, gidx_hbm, dst_hbm, zeros_hbm, part_hbm,
                  acc, gidx_v, dst_a, dst_b, dst_c, rows_a, rows_b, rows_c,
                  dst_t, rows_t,
                  sem_ra, sem_rb, sem_rc, sem_da, sem_db, sem_dc,
                  sem_sa, sem_sb, sem_sc, sem_rt, sem_dt):
        cid = lax.axis_index("c")
        sid = lax.axis_index("s")
        wid = cid * NS + sid
        base = wid * e_per_w

        pltpu.sync_copy(gidx_hbm.at[pl.ds(base, e_per_w)], gidx_v)

        def chunk(g, dst_buf, rows_buf, sem_r, sem_d):
            d_d = pltpu.async_copy(dst_hbm.at[pl.ds(base + g * b, b)], dst_buf, sem_d)
            d_r = pltpu.async_copy(y_hbm.at[gidx_v.at[pl.ds(g * b, b)]], rows_buf, sem_r)
            return d_d, d_r

        in_a0 = chunk(0, dst_a, rows_a, sem_ra, sem_da)
        in_b0 = chunk(1, dst_b, rows_b, sem_rb, sem_db)
        in_c0 = chunk(2, dst_c, rows_c, sem_rc, sem_dc)
        in_t0 = pltpu.async_copy(
            dst_hbm.at[pl.ds(base + ch * b, bt)], dst_t, sem_dt)
        in_t1 = pltpu.async_copy(
            y_hbm.at[gidx_v.at[pl.ds(ch * b, bt)]], rows_t, sem_rt)

        @pl.when(sid < NS - 1)
        def _():
            pltpu.sync_copy(zeros_hbm.at[pl.ds(0, r8)], acc.at[pl.ds(sid * r8, r8)])

        @pl.when(sid == NS - 1)
        def _():
            pltpu.sync_copy(zeros_hbm, acc.at[pl.ds((NS - 1) * r8, r_last)])

        plsc.subcore_barrier()

        del in_a0, in_b0, in_c0, in_t0, in_t1

        def wait_gather(g, dst_buf, rows_buf, sem_r, sem_d):
            pltpu.make_async_copy(
                y_hbm.at[gidx_v.at[pl.ds(g * b, b)]], rows_buf, sem_r).wait()
            pltpu.make_async_copy(
                dst_hbm.at[pl.ds(base + g * b, b)], dst_buf, sem_d).wait()

        nt = ch // 3

        def triple_body(p, carry):
            g = p * 3
            wait_gather(g, dst_a, rows_a, sem_ra, sem_da)
            s_a = pltpu.async_copy(rows_a, acc.at[dst_a], sem_sa, add=True)
            wait_gather(g + 1, dst_b, rows_b, sem_rb, sem_db)
            s_b = pltpu.async_copy(rows_b, acc.at[dst_b], sem_sb, add=True)
            wait_gather(g + 2, dst_c, rows_c, sem_rc, sem_dc)
            s_c = pltpu.async_copy(rows_c, acc.at[dst_c], sem_sc, add=True)
            s_a.wait()
            chunk(g + 3, dst_a, rows_a, sem_ra, sem_da)
            s_b.wait()
            chunk(g + 4, dst_b, rows_b, sem_rb, sem_db)
            s_c.wait()
            chunk(g + 5, dst_c, rows_c, sem_rc, sem_dc)
            return carry

        lax.fori_loop(0, nt - 1, triple_body, 0)

        gl = (nt - 1) * 3
        wait_gather(gl, dst_a, rows_a, sem_ra, sem_da)
        s_a = pltpu.async_copy(rows_a, acc.at[dst_a], sem_sa, add=True)
        wait_gather(gl + 1, dst_b, rows_b, sem_rb, sem_db)
        s_b = pltpu.async_copy(rows_b, acc.at[dst_b], sem_sb, add=True)
        wait_gather(gl + 2, dst_c, rows_c, sem_rc, sem_dc)
        s_c = pltpu.async_copy(rows_c, acc.at[dst_c], sem_sc, add=True)

        slots = [
            (dst_a, rows_a, sem_ra, sem_da, sem_sa, s_a),
            (dst_b, rows_b, sem_rb, sem_db, sem_sb, s_b),
            (dst_c, rows_c, sem_rc, sem_dc, sem_sc, s_c),
        ]
        rem = ch - nt * 3
        for i in range(rem):
            d, r, sr, sd, ss, s = slots[i]
            s.wait()
            chunk(gl + 3 + i, d, r, sr, sd)
        for i in range(rem, 3):
            slots[i][5].wait()
        pend = []
        for i in range(rem):
            d, r, sr, sd, ss, _ = slots[i]
            wait_gather(gl + 3 + i, d, r, sr, sd)
            pend.append(pltpu.async_copy(r, acc.at[d], ss, add=True))
        pltpu.make_async_copy(
            y_hbm.at[gidx_v.at[pl.ds(ch * b, bt)]], rows_t, sem_rt).wait()
        pltpu.make_async_copy(
            dst_hbm.at[pl.ds(base + ch * b, bt)], dst_t, sem_dt).wait()
        pltpu.sync_copy(rows_t, acc.at[dst_t], add=True)
        for s in pend:
            s.wait()
        plsc.subcore_barrier()

        @pl.when(sid < NS - 1)
        def _():
            pltpu.sync_copy(
                acc.at[pl.ds(sid * r8, r8)],
                part_hbm.at[cid].at[pl.ds(sid * r8, r8)],
            )

        @pl.when(sid == NS - 1)
        def _():
            pltpu.sync_copy(
                acc.at[pl.ds((NS - 1) * r8, r_last)],
                part_hbm.at[cid].at[pl.ds((NS - 1) * r8, r_last)],
            )

    return sc_kernel


def _combine_bn_body(n, p_ref, g_ref, b_ref, o_ref):
    s = p_ref[0] + p_ref[1]
    mean = jnp.sum(s, axis=0, keepdims=True) / n
    var = jnp.sum(s * s, axis=0, keepdims=True) / n - mean * mean
    scale = lax.rsqrt(var + BN_EPS) * g_ref[...]
    o_ref[...] = (s - mean) * scale + b_ref[...]


def _combine_bn(part, gamma, beta):
    _, n, c = part.shape
    return pl.pallas_call(
        functools.partial(_combine_bn_body, n),
        in_specs=[
            pl.BlockSpec((2, n, c), lambda: (0, 0, 0)),
            pl.BlockSpec((1, c), lambda: (0, 0)),
            pl.BlockSpec((1, c), lambda: (0, 0)),
        ],
        out_specs=pl.BlockSpec((n, c), lambda: (0, 0)),
        out_shape=jax.ShapeDtypeStruct((n, c), jnp.float32),
    )(part, gamma, beta)


@jax.jit
def kernel(x, edge_index, kernel_offsets, W, bn_gamma, bn_beta):
    n, c_in = x.shape
    k, _, c_out = W.shape
    e = edge_index.shape[1]

    b = 96
    e_per_w = e // NW
    ch = e_per_w // b
    bt = e_per_w - ch * b
    r8 = (n // NS) // 8 * 8

    y, gidx, dst, zeros = _transform_features(
        x, W, edge_index, kernel_offsets, n - (NS - 1) * r8)

    part = _make_sc_edge_kernel(n, c_out, e, ch, b, bt)(y, gidx, dst, zeros)

    return _combine_bn(part, bn_gamma.reshape(1, c_out),
                       bn_beta.reshape(1, c_out))


# speedup vs baseline: 1.2252x; 1.0001x over previous
import functools

import jax
import jax.numpy as jnp
from jax import lax
from jax.experimental import pallas as pl
from jax.experimental.pallas import tpu as pltpu
from jax.experimental.pallas import tpu_sc as plsc

BN_EPS = 1e-5

NC = 2
NS = 16
NW = NC * NS


def _mm_prep_body(n, x_ref, w_ref, ei_ref, ko_ref, y_ref, g_ref, d_ref, z_ref):
    y_ref[...] = jnp.dot(x_ref[...], w_ref[0], preferred_element_type=jnp.float32)

    @pl.when(pl.program_id(0) == 0)
    def _():
        g_ref[...] = ko_ref[...] * n + ei_ref[0, :]
        d_ref[...] = ei_ref[1, :]
        z_ref[...] = jnp.zeros_like(z_ref)


def _transform_features(x, W, edge_index, kernel_offsets, zrows):
    n, c_in = x.shape
    k, _, c_out = W.shape
    e = edge_index.shape[1]
    const1 = lambda i: (0,)
    const2 = lambda i: (0, 0)
    return pl.pallas_call(
        functools.partial(_mm_prep_body, n),
        grid=(k,),
        in_specs=[
            pl.BlockSpec((n, c_in), const2),
            pl.BlockSpec((1, c_in, c_out), lambda i: (i, 0, 0)),
            pl.BlockSpec((2, e), const2),
            pl.BlockSpec((e,), const1),
        ],
        out_specs=[
            pl.BlockSpec((n, c_out), lambda i: (i, 0)),
            pl.BlockSpec((e,), const1),
            pl.BlockSpec((e,), const1),
            pl.BlockSpec((zrows, c_out), const2),
        ],
        out_shape=[
            jax.ShapeDtypeStruct((k * n, c_out), jnp.float32),
            jax.ShapeDtypeStruct((e,), jnp.int32),
            jax.ShapeDtypeStruct((e,), jnp.int32),
            jax.ShapeDtypeStruct((zrows, c_out), jnp.float32),
        ],
    )(x, W, edge_index, kernel_offsets)


def _make_sc_edge_kernel(n, c, e, ch, b, bt):
    mesh = plsc.VectorSubcoreMesh(
        core_axis_name="c", subcore_axis_name="s", num_cores=NC, num_subcores=NS
    )
    r8 = (n // NS) // 8 * 8
    r_last = n - (NS - 1) * r8
    e_per_w = e // NW

    @functools.partial(
        pl.kernel,
        mesh=mesh,
        out_type=jax.ShapeDtypeStruct((NC, n, c), jnp.float32),
        scratch_types=[
            pltpu.VMEM_SHARED((n, c), jnp.float32),
            pltpu.VMEM((e_per_w,), jnp.int32),
            pltpu.VMEM((b,), jnp.int32),
            pltpu.VMEM((b,), jnp.int32),
            pltpu.VMEM((b,), jnp.int32),
            pltpu.VMEM((b, c), jnp.float32),
            pltpu.VMEM((b, c), jnp.float32),
            pltpu.VMEM((b, c), jnp.float32),
            pltpu.VMEM((bt,), jnp.int32),
            pltpu.VMEM((bt, c), jnp.float32),
            pltpu.SemaphoreType.DMA,
            pltpu.SemaphoreType.DMA,
            pltpu.SemaphoreType.DMA,
            pltpu.SemaphoreType.DMA,
            pltpu.SemaphoreType.DMA,
            pltpu.SemaphoreType.DMA,
            pltpu.SemaphoreType.DMA,
            pltpu.SemaphoreType.DMA,
            pltpu.SemaphoreType.DMA,
            pltpu.SemaphoreType.DMA,
            pltpu.SemaphoreType.DMA,
        ],
    )
    def sc_kernel(y_hbm, gidx_hbm, dst_hbm, zeros_hbm, part_hbm,
                  acc, gidx_v, dst_a, dst_b, dst_c, rows_a, rows_b, rows_c,
                  dst_t, rows_t,
                  sem_ra, sem_rb, sem_rc, sem_da, sem_db, sem_dc,
                  sem_sa, sem_sb, sem_sc, sem_rt, sem_dt):
        cid = lax.axis_index("c")
        sid = lax.axis_index("s")
        wid = cid * NS + sid
        base = wid * e_per_w

        pltpu.sync_copy(gidx_hbm.at[pl.ds(base, e_per_w)], gidx_v)

        def chunk(g, dst_buf, rows_buf, sem_r, sem_d):
            d_r = pltpu.async_copy(y_hbm.at[gidx_v.at[pl.ds(g * b, b)]], rows_buf, sem_r)
            d_d = pltpu.async_copy(dst_hbm.at[pl.ds(base + g * b, b)], dst_buf, sem_d)
            return d_d, d_r

        in_a0 = chunk(0, dst_a, rows_a, sem_ra, sem_da)
        in_b0 = chunk(1, dst_b, rows_b, sem_rb, sem_db)
        in_c0 = chunk(2, dst_c, rows_c, sem_rc, sem_dc)
        in_t0 = pltpu.async_copy(
            dst_hbm.at[pl.ds(base + ch * b, bt)], dst_t, sem_dt)
        in_t1 = pltpu.async_copy(
            y_hbm.at[gidx_v.at[pl.ds(ch * b, bt)]], rows_t, sem_rt)

        @pl.when(sid < NS - 1)
        def _():
            pltpu.sync_copy(zeros_hbm.at[pl.ds(0, r8)], acc.at[pl.ds(sid * r8, r8)])

        @pl.when(sid == NS - 1)
        def _():
            pltpu.sync_copy(zeros_hbm, acc.at[pl.ds((NS - 1) * r8, r_last)])

        plsc.subcore_barrier()

        del in_a0, in_b0, in_c0, in_t0, in_t1

        def wait_gather(g, dst_buf, rows_buf, sem_r, sem_d):
            pltpu.make_async_copy(
                y_hbm.at[gidx_v.at[pl.ds(g * b, b)]], rows_buf, sem_r).wait()
            pltpu.make_async_copy(
                dst_hbm.at[pl.ds(base + g * b, b)], dst_buf, sem_d).wait()

        nt = ch // 3

        def triple_body(p, carry):
            g = p * 3
            wait_gather(g, dst_a, rows_a, sem_ra, sem_da)
            s_a = pltpu.async_copy(rows_a, acc.at[dst_a], sem_sa, add=True)
            wait_gather(g + 1, dst_b, rows_b, sem_rb, sem_db)
            s_b = pltpu.async_copy(rows_b, acc.at[dst_b], sem_sb, add=True)
            wait_gather(g + 2, dst_c, rows_c, sem_rc, sem_dc)
            s_c = pltpu.async_copy(rows_c, acc.at[dst_c], sem_sc, add=True)
            s_a.wait()
            chunk(g + 3, dst_a, rows_a, sem_ra, sem_da)
            s_b.wait()
            chunk(g + 4, dst_b, rows_b, sem_rb, sem_db)
            s_c.wait()
            chunk(g + 5, dst_c, rows_c, sem_rc, sem_dc)
            return carry

        lax.fori_loop(0, nt - 1, triple_body, 0)

        gl = (nt - 1) * 3
        wait_gather(gl, dst_a, rows_a, sem_ra, sem_da)
        s_a = pltpu.async_copy(rows_a, acc.at[dst_a], sem_sa, add=True)
        wait_gather(gl + 1, dst_b, rows_b, sem_rb, sem_db)
        s_b = pltpu.async_copy(rows_b, acc.at[dst_b], sem_sb, add=True)
        wait_gather(gl + 2, dst_c, rows_c, sem_rc, sem_dc)
        s_c = pltpu.async_copy(rows_c, acc.at[dst_c], sem_sc, add=True)

        slots = [
            (dst_a, rows_a, sem_ra, sem_da, sem_sa, s_a),
            (dst_b, rows_b, sem_rb, sem_db, sem_sb, s_b),
            (dst_c, rows_c, sem_rc, sem_dc, sem_sc, s_c),
        ]
        rem = ch - nt * 3
        for i in range(rem):
            d, r, sr, sd, ss, s = slots[i]
            s.wait()
            chunk(gl + 3 + i, d, r, sr, sd)
        for i in range(rem, 3):
            slots[i][5].wait()
        pend = []
        for i in range(rem):
            d, r, sr, sd, ss, _ = slots[i]
            wait_gather(gl + 3 + i, d, r, sr, sd)
            pend.append(pltpu.async_copy(r, acc.at[d], ss, add=True))
        pltpu.make_async_copy(
            y_hbm.at[gidx_v.at[pl.ds(ch * b, bt)]], rows_t, sem_rt).wait()
        pltpu.make_async_copy(
            dst_hbm.at[pl.ds(base + ch * b, bt)], dst_t, sem_dt).wait()
        pltpu.sync_copy(rows_t, acc.at[dst_t], add=True)
        for s in pend:
            s.wait()
        plsc.subcore_barrier()

        @pl.when(sid < NS - 1)
        def _():
            pltpu.sync_copy(
                acc.at[pl.ds(sid * r8, r8)],
                part_hbm.at[cid].at[pl.ds(sid * r8, r8)],
            )

        @pl.when(sid == NS - 1)
        def _():
            pltpu.sync_copy(
                acc.at[pl.ds((NS - 1) * r8, r_last)],
                part_hbm.at[cid].at[pl.ds((NS - 1) * r8, r_last)],
            )

    return sc_kernel


def _combine_bn_body(n, p_ref, g_ref, b_ref, o_ref):
    s = p_ref[0] + p_ref[1]
    mean = jnp.sum(s, axis=0, keepdims=True) / n
    var = jnp.sum(s * s, axis=0, keepdims=True) / n - mean * mean
    scale = lax.rsqrt(var + BN_EPS) * g_ref[...]
    o_ref[...] = (s - mean) * scale + b_ref[...]


def _combine_bn(part, gamma, beta):
    _, n, c = part.shape
    return pl.pallas_call(
        functools.partial(_combine_bn_body, n),
        in_specs=[
            pl.BlockSpec((2, n, c), lambda: (0, 0, 0)),
            pl.BlockSpec((1, c), lambda: (0, 0)),
            pl.BlockSpec((1, c), lambda: (0, 0)),
        ],
        out_specs=pl.BlockSpec((n, c), lambda: (0, 0)),
        out_shape=jax.ShapeDtypeStruct((n, c), jnp.float32),
    )(part, gamma, beta)


@jax.jit
def kernel(x, edge_index, kernel_offsets, W, bn_gamma, bn_beta):
    n, c_in = x.shape
    k, _, c_out = W.shape
    e = edge_index.shape[1]

    b = 96
    e_per_w = e // NW
    ch = e_per_w // b
    bt = e_per_w - ch * b
    r8 = (n // NS) // 8 * 8

    y, gidx, dst, zeros = _transform_features(
        x, W, edge_index, kernel_offsets, n - (NS - 1) * r8)

    part = _make_sc_edge_kernel(n, c_out, e, ch, b, bt)(y, gidx, dst, zeros)

    return _combine_bn(part, bn_gamma.reshape(1, c_out),
                       bn_beta.reshape(1, c_out))
